# Initial kernel scaffold; baseline (speedup 1.0000x reference)
import jax
import jax.numpy as jnp
from jax.experimental import pallas as pl


def kernel(input_ids, word_embeddings, token_type_embeddings, position_embeddings, ln_gamma, ln_beta):
    raise NotImplementedError("write your pallas kernel here")


# speedup vs baseline: 3.9624x; 3.9624x over previous
import functools

import jax
import jax.numpy as jnp
from jax import lax
from jax.experimental import pallas as pl
from jax.experimental.pallas import tpu as pltpu
from jax.experimental.pallas import tpu_sc as plsc

B, L, HID = 1024, 200, 128
MAX_POS = 512
PAD_IDX = 0
EPS = 1e-12
N = B * L

NC, NS, LANES = 2, 16, 16
NW = NC * NS
TPW = N // NW
CHUNK = 128
NCHUNKS = TPW // CHUNK
UNROLL = 4
NVEC = HID // LANES


def _sc_body(ids_hbm, pids_hbm, word_hbm, ptt_hbm, gb_hbm, out_hbm,
             ids_v, pids_v, ptt_v, gb_v, buf_v, sem):
    wid = lax.axis_index("s") * NC + lax.axis_index("c")
    base = wid * TPW

    pltpu.sync_copy(ids_hbm.at[pl.ds(base, TPW)], ids_v)
    pltpu.sync_copy(pids_hbm.at[pl.ds(base, TPW)], pids_v)
    pltpu.sync_copy(ptt_hbm, ptt_v)
    pltpu.sync_copy(gb_hbm, gb_v)

    iota = lax.iota(jnp.int32, LANES)
    gammas = [gb_v[0, pl.ds(LANES * j, LANES)] for j in range(NVEC)]
    betas = [gb_v[1, pl.ds(LANES * j, LANES)] for j in range(NVEC)]

    def ln_token(t, cbase):
        pid = plsc.load_gather(pids_v, [jnp.full((LANES,), cbase + t, jnp.int32)])
        xs = []
        s = jnp.zeros((LANES,), jnp.float32)
        q = jnp.zeros((LANES,), jnp.float32)
        for j in range(NVEC):
            w = buf_v[t, pl.ds(LANES * j, LANES)]
            p = plsc.load_gather(ptt_v, [pid, iota + (LANES * j)])
            x = w + p
            xs.append(x)
            s = s + x
            q = q + x * x
        mean = jnp.sum(s) * (1.0 / HID)
        var = jnp.sum(q) * (1.0 / HID) - mean * mean
        vv = jnp.full((LANES,), var + EPS)
        yi = lax.bitcast_convert_type(vv, jnp.int32)
        y = lax.bitcast_convert_type(
            jnp.int32(0x5F3759DF) - lax.shift_right_logical(yi, 1), jnp.float32)
        for _ in range(3):
            y = y * (1.5 - 0.5 * vv * y * y)
        mvec = jnp.full((LANES,), mean)
        for j in range(NVEC):
            gj = gammas[j] * y
            bj = betas[j] - mvec * gj
            buf_v[t, pl.ds(LANES * j, LANES)] = xs[j] * gj + bj

    def chunk_body(g, carry):
        cbase = g * CHUNK
        pltpu.async_copy(word_hbm.at[ids_v.at[pl.ds(cbase, CHUNK)]], buf_v, sem).wait()

        def tok_group(i, c):
            for u in range(UNROLL):
                ln_token(i * UNROLL + u, cbase)
            return c

        lax.fori_loop(0, CHUNK // UNROLL, tok_group, 0)
        pltpu.sync_copy(buf_v, out_hbm.at[pl.ds(base + cbase, CHUNK)])
        return carry

    lax.fori_loop(0, NCHUNKS, chunk_body, 0)


@jax.jit
def _sc_call(ids_flat, pids_flat, word_embeddings, ptt, gb):
    mesh = plsc.VectorSubcoreMesh(core_axis_name="c", subcore_axis_name="s")
    f = functools.partial(
        pl.kernel,
        mesh=mesh,
        compiler_params=pltpu.CompilerParams(needs_layout_passes=False),
        out_type=jax.ShapeDtypeStruct((N, HID), jnp.float32),
        scratch_types=[
            pltpu.VMEM((TPW,), jnp.int32),
            pltpu.VMEM((TPW,), jnp.int32),
            pltpu.VMEM((MAX_POS, HID), jnp.float32),
            pltpu.VMEM((2, HID), jnp.float32),
            pltpu.VMEM((CHUNK, HID), jnp.float32),
            pltpu.SemaphoreType.DMA,
        ],
    )(_sc_body)
    return f(ids_flat, pids_flat, word_embeddings, ptt, gb)


def kernel(input_ids, word_embeddings, token_type_embeddings, position_embeddings,
           ln_gamma, ln_beta):
    ids = input_ids.astype(jnp.int32)
    mask = (ids != PAD_IDX).astype(jnp.int32)
    pos_ids = jnp.cumsum(mask, axis=1) * mask + PAD_IDX
    ptt = position_embeddings + token_type_embeddings[0]
    gb = jnp.stack([ln_gamma, ln_beta])
    out = _sc_call(ids.reshape(-1), pos_ids.reshape(-1), word_embeddings, ptt, gb)
    return out.reshape(B, L, HID)


# speedup vs baseline: 4.9560x; 1.2507x over previous
import functools

import jax
import jax.numpy as jnp
from jax import lax
from jax.experimental import pallas as pl
from jax.experimental.pallas import tpu as pltpu
from jax.experimental.pallas import tpu_sc as plsc

B, L, HID = 1024, 200, 128
MAX_POS = 512
PAD_IDX = 0
EPS = 1e-12
N = B * L

NC, NS, LANES = 2, 16, 16
NW = NC * NS
TPW = N // NW
CHUNK = 64
NCHUNKS = TPW // CHUNK
NBUF = 4
UNROLL = 4
NVEC = HID // LANES

assert (NCHUNKS - 4) % NBUF == 0


def _sc_body(ids_hbm, pids_hbm, word_hbm, ptt_hbm, gb_hbm, out_hbm,
             ids_v, pids_v, ptt_v, gb_v,
             buf0_v, buf1_v, buf2_v, buf3_v,
             sg0, sg1, sg2, sg3, so0, so1, so2, so3):
    wid = lax.axis_index("s") * NC + lax.axis_index("c")
    base = wid * TPW

    pltpu.sync_copy(ids_hbm.at[pl.ds(base, TPW)], ids_v)
    pltpu.sync_copy(pids_hbm.at[pl.ds(base, TPW)], pids_v)
    pltpu.sync_copy(ptt_hbm, ptt_v)
    pltpu.sync_copy(gb_hbm, gb_v)

    bufs = (buf0_v, buf1_v, buf2_v, buf3_v)
    sems_g = (sg0, sg1, sg2, sg3)
    sems_o = (so0, so1, so2, so3)

    iota = lax.iota(jnp.int32, LANES)
    gammas = [gb_v[0, pl.ds(LANES * j, LANES)] for j in range(NVEC)]
    betas = [gb_v[1, pl.ds(LANES * j, LANES)] for j in range(NVEC)]

    def start_gather(g, b):
        pltpu.async_copy(
            word_hbm.at[ids_v.at[pl.ds(g * CHUNK, CHUNK)]], bufs[b], sems_g[b])

    def wait_gather(b):
        pltpu.make_async_copy(
            word_hbm.at[ids_v.at[pl.ds(0, CHUNK)]], bufs[b], sems_g[b]).wait()

    def start_out(g, b):
        pltpu.async_copy(
            bufs[b], out_hbm.at[pl.ds(base + g * CHUNK, CHUNK)], sems_o[b])

    def wait_out(b):
        pltpu.make_async_copy(
            bufs[b], out_hbm.at[pl.ds(base, CHUNK)], sems_o[b]).wait()

    def ln_token(t, cbase, buf):
        pid = plsc.load_gather(pids_v, [jnp.full((LANES,), cbase + t, jnp.int32)])
        xs = []
        s = jnp.zeros((LANES,), jnp.float32)
        q = jnp.zeros((LANES,), jnp.float32)
        for j in range(NVEC):
            w = buf[t, pl.ds(LANES * j, LANES)]
            p = plsc.load_gather(ptt_v, [pid, iota + (LANES * j)])
            x = w + p
            xs.append(x)
            s = s + x
            q = q + x * x
        mean = jnp.sum(s) * (1.0 / HID)
        var = jnp.sum(q) * (1.0 / HID) - mean * mean
        vv = jnp.full((LANES,), var + EPS)
        yi = lax.bitcast_convert_type(vv, jnp.int32)
        y = lax.bitcast_convert_type(
            jnp.int32(0x5F3759DF) - lax.shift_right_logical(yi, 1), jnp.float32)
        for _ in range(3):
            y = y * (1.5 - 0.5 * vv * y * y)
        my = jnp.full((LANES,), mean) * y
        for j in range(NVEC):
            u = xs[j] * y - my
            buf[t, pl.ds(LANES * j, LANES)] = u * gammas[j] + betas[j]

    def compute(g, b):
        cbase = g * CHUNK
        buf = bufs[b]

        def tok_group(i, c):
            for u in range(UNROLL):
                ln_token(i * UNROLL + u, cbase, buf)
            return c

        lax.fori_loop(0, CHUNK // UNROLL, tok_group, 0)

    start_gather(0, 0)
    start_gather(1, 1)
    wait_gather(0)
    start_gather(2, 2)
    compute(0, 0)
    start_out(0, 0)
    wait_gather(1)
    start_gather(3, 3)
    compute(1, 1)
    start_out(1, 1)

    def ring_body(p, c):
        for q in range(NBUF):
            g = NBUF * p + 2 + q
            b = (2 + q) % NBUF
            bn = (b + 2) % NBUF
            wait_gather(b)
            wait_out(bn)
            start_gather(g + 2, bn)
            compute(g, b)
            start_out(g, b)
        return c

    lax.fori_loop(0, (NCHUNKS - 4) // NBUF, ring_body, 0)

    g = NCHUNKS - 2
    b = g % NBUF
    wait_gather(b)
    compute(g, b)
    start_out(g, b)
    g = NCHUNKS - 1
    b = g % NBUF
    wait_gather(b)
    compute(g, b)
    start_out(g, b)
    for b in range(NBUF):
        wait_out(b)


@jax.jit
def _sc_call(ids_flat, pids_flat, word_embeddings, ptt, gb):
    mesh = plsc.VectorSubcoreMesh(core_axis_name="c", subcore_axis_name="s")
    f = functools.partial(
        pl.kernel,
        mesh=mesh,
        compiler_params=pltpu.CompilerParams(needs_layout_passes=False),
        out_type=jax.ShapeDtypeStruct((N, HID), jnp.float32),
        scratch_types=[
            pltpu.VMEM((TPW,), jnp.int32),
            pltpu.VMEM((TPW,), jnp.int32),
            pltpu.VMEM((MAX_POS, HID), jnp.float32),
            pltpu.VMEM((2, HID), jnp.float32),
            pltpu.VMEM((CHUNK, HID), jnp.float32),
            pltpu.VMEM((CHUNK, HID), jnp.float32),
            pltpu.VMEM((CHUNK, HID), jnp.float32),
            pltpu.VMEM((CHUNK, HID), jnp.float32),
            pltpu.SemaphoreType.DMA,
            pltpu.SemaphoreType.DMA,
            pltpu.SemaphoreType.DMA,
            pltpu.SemaphoreType.DMA,
            pltpu.SemaphoreType.DMA,
            pltpu.SemaphoreType.DMA,
            pltpu.SemaphoreType.DMA,
            pltpu.SemaphoreType.DMA,
        ],
    )(_sc_body)
    return f(ids_flat, pids_flat, word_embeddings, ptt, gb)


def kernel(input_ids, word_embeddings, token_type_embeddings, position_embeddings,
           ln_gamma, ln_beta):
    ids = input_ids.astype(jnp.int32)
    mask = (ids != PAD_IDX).astype(jnp.int32)
    pos_ids = jnp.cumsum(mask, axis=1) * mask + PAD_IDX
    ptt = position_embeddings + token_type_embeddings[0]
    gb = jnp.stack([ln_gamma, ln_beta])
    out = _sc_call(ids.reshape(-1), pos_ids.reshape(-1), word_embeddings, ptt, gb)
    return out.reshape(B, L, HID)


# speedup vs baseline: 5.2320x; 1.0557x over previous
import functools

import jax
import jax.numpy as jnp
from jax import lax
from jax.experimental import pallas as pl
from jax.experimental.pallas import tpu as pltpu
from jax.experimental.pallas import tpu_sc as plsc

B, L, HID = 1024, 200, 128
MAX_POS = 512
PAD_IDX = 0
EPS = 1e-12
N = B * L

NC, NS, LANES = 2, 16, 16
NW = NC * NS
TPW = N // NW
CHUNK = 64
NCHUNKS = TPW // CHUNK
NBUF = 4
UNROLL = 8
NVEC = HID // LANES

assert (NCHUNKS - 4) % NBUF == 0


def _sc_body(ids_hbm, pids_hbm, word_hbm, ptt_hbm, gb_hbm, out_hbm,
             ids_v, pids_v, ptt_v, gb_v,
             buf0_v, buf1_v, buf2_v, buf3_v,
             sg0, sg1, sg2, sg3, so0, so1, so2, so3):
    wid = lax.axis_index("s") * NC + lax.axis_index("c")
    base = wid * TPW

    pltpu.sync_copy(ids_hbm.at[pl.ds(base, TPW)], ids_v)
    pltpu.sync_copy(pids_hbm.at[pl.ds(base, TPW)], pids_v)
    pltpu.sync_copy(ptt_hbm, ptt_v)
    pltpu.sync_copy(gb_hbm, gb_v)

    bufs = (buf0_v, buf1_v, buf2_v, buf3_v)
    sems_g = (sg0, sg1, sg2, sg3)
    sems_o = (so0, so1, so2, so3)

    iota = lax.iota(jnp.int32, LANES)
    gammas = [gb_v[0, pl.ds(LANES * j, LANES)] for j in range(NVEC)]
    betas = [gb_v[1, pl.ds(LANES * j, LANES)] for j in range(NVEC)]

    def start_gather(g, b):
        pltpu.async_copy(
            word_hbm.at[ids_v.at[pl.ds(g * CHUNK, CHUNK)]], bufs[b], sems_g[b])

    def wait_gather(b):
        pltpu.make_async_copy(
            word_hbm.at[ids_v.at[pl.ds(0, CHUNK)]], bufs[b], sems_g[b]).wait()

    def start_out(g, b):
        pltpu.async_copy(
            bufs[b], out_hbm.at[pl.ds(base + g * CHUNK, CHUNK)], sems_o[b])

    def wait_out(b):
        pltpu.make_async_copy(
            bufs[b], out_hbm.at[pl.ds(base, CHUNK)], sems_o[b]).wait()

    def ln_token(t, cbase, buf):
        pid = plsc.load_gather(pids_v, [jnp.full((LANES,), cbase + t, jnp.int32)])
        xs = []
        s = jnp.zeros((LANES,), jnp.float32)
        q = jnp.zeros((LANES,), jnp.float32)
        for j in range(NVEC):
            w = buf[t, pl.ds(LANES * j, LANES)]
            p = plsc.load_gather(ptt_v, [pid, iota + (LANES * j)])
            x = w + p
            xs.append(x)
            s = s + x
            q = q + x * x
        mean = jnp.sum(s) * (1.0 / HID)
        var = jnp.sum(q) * (1.0 / HID) - mean * mean
        vv = jnp.full((LANES,), var + EPS)
        yi = lax.bitcast_convert_type(vv, jnp.int32)
        y = lax.bitcast_convert_type(
            jnp.int32(0x5F3759DF) - lax.shift_right_logical(yi, 1), jnp.float32)
        for _ in range(2):
            y = y * (1.5 - 0.5 * vv * y * y)
        my = jnp.full((LANES,), mean) * y
        for j in range(NVEC):
            u = xs[j] * y - my
            buf[t, pl.ds(LANES * j, LANES)] = u * gammas[j] + betas[j]

    def compute(g, b):
        cbase = g * CHUNK
        buf = bufs[b]

        def tok_group(i, c):
            for u in range(UNROLL):
                ln_token(i * UNROLL + u, cbase, buf)
            return c

        lax.fori_loop(0, CHUNK // UNROLL, tok_group, 0)

    start_gather(0, 0)
    start_gather(1, 1)
    wait_gather(0)
    start_gather(2, 2)
    compute(0, 0)
    start_out(0, 0)
    wait_gather(1)
    start_gather(3, 3)
    compute(1, 1)
    start_out(1, 1)

    def ring_body(p, c):
        for q in range(NBUF):
            g = NBUF * p + 2 + q
            b = (2 + q) % NBUF
            bn = (b + 2) % NBUF
            wait_gather(b)
            wait_out(bn)
            start_gather(g + 2, bn)
            compute(g, b)
            start_out(g, b)
        return c

    lax.fori_loop(0, (NCHUNKS - 4) // NBUF, ring_body, 0)

    g = NCHUNKS - 2
    b = g % NBUF
    wait_gather(b)
    compute(g, b)
    start_out(g, b)
    g = NCHUNKS - 1
    b = g % NBUF
    wait_gather(b)
    compute(g, b)
    start_out(g, b)
    for b in range(NBUF):
        wait_out(b)


@jax.jit
def _sc_call(ids_flat, pids_flat, word_embeddings, ptt, gb):
    mesh = plsc.VectorSubcoreMesh(core_axis_name="c", subcore_axis_name="s")
    f = functools.partial(
        pl.kernel,
        mesh=mesh,
        compiler_params=pltpu.CompilerParams(needs_layout_passes=False),
        out_type=jax.ShapeDtypeStruct((N, HID), jnp.float32),
        scratch_types=[
            pltpu.VMEM((TPW,), jnp.int32),
            pltpu.VMEM((TPW,), jnp.int32),
            pltpu.VMEM((MAX_POS, HID), jnp.float32),
            pltpu.VMEM((2, HID), jnp.float32),
            pltpu.VMEM((CHUNK, HID), jnp.float32),
            pltpu.VMEM((CHUNK, HID), jnp.float32),
            pltpu.VMEM((CHUNK, HID), jnp.float32),
            pltpu.VMEM((CHUNK, HID), jnp.float32),
            pltpu.SemaphoreType.DMA,
            pltpu.SemaphoreType.DMA,
            pltpu.SemaphoreType.DMA,
            pltpu.SemaphoreType.DMA,
            pltpu.SemaphoreType.DMA,
            pltpu.SemaphoreType.DMA,
            pltpu.SemaphoreType.DMA,
            pltpu.SemaphoreType.DMA,
        ],
    )(_sc_body)
    return f(ids_flat, pids_flat, word_embeddings, ptt, gb)


def kernel(input_ids, word_embeddings, token_type_embeddings, position_embeddings,
           ln_gamma, ln_beta):
    ids = input_ids.astype(jnp.int32)
    mask = (ids != PAD_IDX).astype(jnp.int32)
    pos_ids = jnp.cumsum(mask, axis=1) * mask + PAD_IDX
    ptt = position_embeddings + token_type_embeddings[0]
    gb = jnp.stack([ln_gamma, ln_beta])
    out = _sc_call(ids.reshape(-1), pos_ids.reshape(-1), word_embeddings, ptt, gb)
    return out.reshape(B, L, HID)
